# baseline (speedup 1.0000x reference)
import jax
import jax.numpy as jnp
from jax import lax
from jax.experimental import pallas as pl
from jax.experimental.pallas import tpu as pltpu

_N = 5000
_E = 80000
_B = 640
_NBLK = _E // _B


def _gib_kernel(x_ref, src_ref, dst_ref, W1_ref, b1_ref, W2_ref, b2_ref,
                Wf1_ref, bf1_ref, Wf2_ref, bf2_ref,
                emb_ref, pos_ref, neg_ref, pen_ref,
                hw_ref, acc_ref, h2w_ref, acc2_ref, deg_ref, asg_ref):
    f32 = jnp.float32

    def onehot_t(ref, i):
        blk = ref[:, pl.ds(i * _B, _B)]
        rows = lax.broadcasted_iota(jnp.int32, (_N, _B), 0)
        return (rows == blk).astype(f32)

    cdims = (((0,), (0,)), ((), ()))

    deg_ref[...] = jnp.ones((_N, 1), f32)

    def deg_body(i, c):
        oh_d = onehot_t(dst_ref, i)
        deg_ref[...] = deg_ref[...] + jnp.dot(
            oh_d, jnp.ones((_B, 1), f32), preferred_element_type=f32)
        return c

    lax.fori_loop(0, _NBLK, deg_body, 0)
    deg_ref[...] = deg_ref[...] ** -0.5

    def conv(hw, acc):
        dinv = deg_ref[...]
        acc[...] = (dinv * dinv) * hw[...]

        def body(i, c):
            oh_s = onehot_t(src_ref, i)
            oh_d = onehot_t(dst_ref, i)
            cs = lax.dot_general(oh_s, deg_ref[...], cdims,
                                 preferred_element_type=f32)
            cd = lax.dot_general(oh_d, deg_ref[...], cdims,
                                 preferred_element_type=f32)
            g = lax.dot_general(oh_s, hw[...], cdims,
                                preferred_element_type=f32)
            m = (cs * cd) * g
            acc[...] = acc[...] + jnp.dot(oh_d, m,
                                          preferred_element_type=f32)
            return c

        lax.fori_loop(0, _NBLK, body, 0)

    hw_ref[...] = jnp.dot(x_ref[...], W1_ref[...], preferred_element_type=f32)
    conv(hw_ref, acc_ref)
    h1 = jnp.maximum(acc_ref[...] + b1_ref[...], 0.0)

    h2w_ref[...] = jnp.dot(h1, W2_ref[...], preferred_element_type=f32)
    conv(h2w_ref, acc2_ref)
    acc2_ref[...] = acc2_ref[...] + b2_ref[...]
    h2 = acc2_ref[...]

    a1 = jnp.tanh(jnp.dot(h2, Wf1_ref[...], preferred_element_type=f32)
                  + bf1_ref[...])
    a2 = jnp.dot(a1, Wf2_ref[...], preferred_element_type=f32) + bf2_ref[...]
    a2 = a2 - jnp.max(a2, axis=1, keepdims=True)
    ea = jnp.exp(a2)
    asg_ref[...] = ea / jnp.sum(ea, axis=1, keepdims=True)

    def adj_body(i, na):
        oh_s = onehot_t(src_ref, i)
        oh_d = onehot_t(dst_ref, i)
        a_s = lax.dot_general(oh_s, asg_ref[...], cdims,
                              preferred_element_type=f32)
        a_d = lax.dot_general(oh_d, asg_ref[...], cdims,
                              preferred_element_type=f32)
        return na + lax.dot_general(a_s, a_d, cdims,
                                    preferred_element_type=f32)

    new_adj = lax.fori_loop(0, _NBLK, adj_body, jnp.zeros((2, 2), f32))

    group = lax.dot_general(asg_ref[...], h2, cdims,
                            preferred_element_type=f32)
    pos_ref[...] = jnp.clip(group[0:1, :], -100.0, 100.0)
    neg_ref[...] = jnp.clip(group[1:2, :], -100.0, 100.0)
    emb_ref[...] = jnp.mean(group, axis=0, keepdims=True)

    denom = jnp.maximum(jnp.sum(jnp.abs(new_adj), axis=1, keepdims=True),
                        1e-12)
    nadj = new_adj / denom
    d0 = nadj[0:1, 0:1]
    d1 = nadj[1:2, 1:2]
    pen_ref[...] = 0.5 * ((d0 - 1.0) ** 2 + (d1 - 1.0) ** 2)


def _gib_call(x, src, dst, W1, b1, W2, b2, Wf1, bf1, Wf2, bf2):
    f32 = jnp.float32
    out_shape = [
        jax.ShapeDtypeStruct((1, 128), f32),
        jax.ShapeDtypeStruct((1, 128), f32),
        jax.ShapeDtypeStruct((1, 128), f32),
        jax.ShapeDtypeStruct((1, 1), f32),
    ]
    scratch = [
        pltpu.VMEM((_N, 256), f32),
        pltpu.VMEM((_N, 256), f32),
        pltpu.VMEM((_N, 128), f32),
        pltpu.VMEM((_N, 128), f32),
        pltpu.VMEM((_N, 1), f32),
        pltpu.VMEM((_N, 2), f32),
    ]
    return pl.pallas_call(
        _gib_kernel,
        out_shape=out_shape,
        scratch_shapes=scratch,
    )(x, src, dst, W1, b1.reshape(1, -1), W2, b2.reshape(1, -1),
      Wf1, bf1.reshape(1, -1), Wf2, bf2.reshape(1, -1))


def _cls_kernel(data_ref, lab_ref, Wc1_ref, bc1_ref, Wc2_ref, bc2_ref,
                p0_ref, p1_ref, cls_ref, pen_ref):
    f32 = jnp.float32
    h = jnp.maximum(jnp.dot(data_ref[...], Wc1_ref[...],
                            preferred_element_type=f32) + bc1_ref[...], 0.0)
    pred = jnp.maximum(jnp.dot(h, Wc2_ref[...],
                               preferred_element_type=f32) + bc2_ref[...],
                       0.0)
    d = pred - lab_ref[...]
    cls_ref[...] = jnp.sum(d * d) * (1.0 / d.shape[0]) * jnp.ones((1, 1), f32)
    pen_ref[...] = (p0_ref[...] + p1_ref[...]) * (0.5 * 5.0)


def kernel(features, edges, labels, W1, b1, W2, b2, Wf1, bf1, Wf2, bf2,
           Wc1, bc1, Wc2, bc2):
    f32 = jnp.float32
    G = features.shape[0]
    embs, poss, negs, pens = [], [], [], []
    for g in range(G):
        src = edges[g, 0].reshape(1, _E).astype(jnp.int32)
        dst = edges[g, 1].reshape(1, _E).astype(jnp.int32)
        e, p, ng, pp = _gib_call(features[g], src, dst,
                                 W1, b1, W2, b2, Wf1, bf1, Wf2, bf2)
        embs.append(e)
        poss.append(p)
        negs.append(ng)
        pens.append(pp)

    embeddings = jnp.concatenate(embs, axis=0)
    positive = jnp.concatenate(poss, axis=0)
    negative = jnp.concatenate(negs, axis=0)

    data = jnp.concatenate([embeddings, positive], axis=0)
    lab = labels.reshape(-1, 1).astype(f32)
    lab2 = jnp.concatenate([lab, lab], axis=0)
    cls, pen = pl.pallas_call(
        _cls_kernel,
        out_shape=[jax.ShapeDtypeStruct((1, 1), f32),
                   jax.ShapeDtypeStruct((1, 1), f32)],
    )(data, lab2, Wc1, bc1.reshape(1, -1), Wc2, bc2.reshape(1, -1),
      pens[0], pens[1])

    return embeddings, positive, negative, cls[0, 0], pen[0, 0]
